# Initial kernel scaffold; baseline (speedup 1.0000x reference)
import jax
import jax.numpy as jnp
from jax.experimental import pallas as pl


def kernel(x, table):
    raise NotImplementedError("write your pallas kernel here")


# speedup vs baseline: 1.6460x; 1.6460x over previous
import functools

import jax
import jax.numpy as jnp
from jax import lax
from jax.experimental import pallas as pl
from jax.experimental.pallas import tpu as pltpu
from jax.experimental.pallas import tpu_sc as plsc

D = 768
B = 4 * 8192
NC = 2
NS = 16
NW = NC * NS
B_PER_W = B // NW
CHUNK = 64
N_CHUNKS = B_PER_W // CHUNK


def _make_emb():
    mesh = plsc.VectorSubcoreMesh(core_axis_name="c", subcore_axis_name="s")

    @functools.partial(
        pl.kernel,
        mesh=mesh,
        out_type=jax.ShapeDtypeStruct((B, D), jnp.float32),
        scratch_types=[
            pltpu.VMEM((N_CHUNKS, CHUNK), jnp.int32),
            pltpu.VMEM((CHUNK, D), jnp.float32),
            pltpu.VMEM((CHUNK, D), jnp.float32),
            pltpu.SemaphoreType.DMA,
            pltpu.SemaphoreType.DMA,
        ],
    )
    def emb(table_hbm, idx_hbm, out_hbm, idx_v, rows0, rows1, sem0, sem1):
        wid = lax.axis_index("s") * NC + lax.axis_index("c")
        base = wid * B_PER_W
        pltpu.sync_copy(idx_hbm.at[wid], idx_v)
        bufs = (rows0, rows1)
        sems = (sem0, sem1)
        copies = [None, None]
        copies[0] = pltpu.async_copy(table_hbm.at[idx_v.at[0]], rows0, sem0)
        for j in range(N_CHUNKS):
            b = j % 2
            copies[b].wait()
            if j + 1 < N_CHUNKS:
                nb = (j + 1) % 2
                copies[nb] = pltpu.async_copy(
                    table_hbm.at[idx_v.at[j + 1]], bufs[nb], sems[nb])
            pltpu.sync_copy(bufs[b], out_hbm.at[pl.ds(base + j * CHUNK, CHUNK)])

    return emb


_emb = _make_emb()


@jax.jit
def kernel(x, table):
    ids = x.astype(jnp.int32).reshape(NW, N_CHUNKS, CHUNK)
    out = _emb(table, ids)
    return out.reshape(x.shape[0], x.shape[1], D)


# speedup vs baseline: 1.6820x; 1.0219x over previous
import functools

import jax
import jax.numpy as jnp
from jax import lax
from jax.experimental import pallas as pl
from jax.experimental.pallas import tpu as pltpu
from jax.experimental.pallas import tpu_sc as plsc

D = 768
B = 4 * 8192
NC = 2
NS = 16
NW = NC * NS
B_PER_W = B // NW
CHUNK = 32
N_CHUNKS = B_PER_W // CHUNK
NBUF = 5


def _make_emb():
    mesh = plsc.VectorSubcoreMesh(core_axis_name="c", subcore_axis_name="s")

    @functools.partial(
        pl.kernel,
        mesh=mesh,
        out_type=jax.ShapeDtypeStruct((B, D), jnp.float32),
        scratch_types=(
            [pltpu.VMEM((N_CHUNKS, CHUNK), jnp.int32)]
            + [pltpu.VMEM((CHUNK, D), jnp.float32)] * NBUF
            + [pltpu.SemaphoreType.DMA] * (2 * NBUF)
        ),
    )
    def emb(table_hbm, idx_hbm, out_hbm, idx_v, *rest):
        bufs = rest[:NBUF]
        gsems = rest[NBUF:2 * NBUF]
        wsems = rest[2 * NBUF:]
        wid = lax.axis_index("s") * NC + lax.axis_index("c")
        base = wid * B_PER_W
        pltpu.sync_copy(idx_hbm.at[wid], idx_v)
        gcop = [None] * NBUF
        wcop = [None] * NBUF
        for b in range(NBUF):
            gcop[b] = pltpu.async_copy(table_hbm.at[idx_v.at[b]], bufs[b], gsems[b])
        for j in range(N_CHUNKS):
            b = j % NBUF
            gcop[b].wait()
            wcop[b] = pltpu.async_copy(
                bufs[b], out_hbm.at[pl.ds(base + j * CHUNK, CHUNK)], wsems[b])
            nj = j + NBUF
            if nj < N_CHUNKS:
                wcop[b].wait()
                gcop[b] = pltpu.async_copy(
                    table_hbm.at[idx_v.at[nj]], bufs[b], gsems[b])
        for j in range(max(0, N_CHUNKS - NBUF), N_CHUNKS):
            wcop[j % NBUF].wait()

    return emb


_emb = _make_emb()


@jax.jit
def kernel(x, table):
    ids = x.astype(jnp.int32).reshape(NW, N_CHUNKS, CHUNK)
    out = _emb(table, ids)
    return out.reshape(x.shape[0], x.shape[1], D)
